# Initial kernel scaffold; baseline (speedup 1.0000x reference)
import jax
import jax.numpy as jnp
from jax.experimental import pallas as pl


def kernel(x, edge_index, W1, b1, g1, be1, W2, b2, g2, be2, W3, b3, g3, be3, fW1, fb1, fW2, fb2):
    raise NotImplementedError("write your pallas kernel here")


# speedup vs baseline: 12.8499x; 12.8499x over previous
import functools

import jax
import jax.numpy as jnp
from jax import lax
from jax.experimental import pallas as pl
from jax.experimental.pallas import tpu as pltpu
from jax.experimental.pallas import tpu_sc as plsc

_N = 10000
_E = 320000
_D = 128
_H = 128

_NC = 2
_NS = 16

_CHUNK = 128
_NCHUNK = _E // _CHUNK
_CPC = _NCHUNK // _NC
_FULL_IT = _CPC // _NS
_EXTRA = _CPC - _NS * _FULL_IT

_WCH = 80
_NWCH = _N // _WCH
_W_FULL = _NWCH // _NS
_W_EXTRA = _NWCH - _NS * _W_FULL

_sc_mesh = plsc.VectorSubcoreMesh(core_axis_name="c", subcore_axis_name="s")


def _chunk_offset(cid, sid, k):
    return (cid * _CPC + sid + _NS * k) * _CHUNK


def _row0(sid, i):
    return (sid + _NS * i) * _WCH


@functools.partial(
    pl.kernel,
    out_type=jax.ShapeDtypeStruct((_NC, _N, _H), jnp.float32),
    mesh=_sc_mesh,
    scratch_types=[
        pltpu.VMEM((_CHUNK,), jnp.int32),
        pltpu.VMEM((_CHUNK, _H), jnp.float32),
        pltpu.VMEM((_WCH, _H), jnp.float32),
        pltpu.VMEM_SHARED((_N, _H), jnp.float32),
    ],
)
def _deg_kernel(dst_hbm, out_hbm, didx, ones, bounce, acc):
    cid = lax.axis_index("c")
    sid = lax.axis_index("s")

    def fill_ones(i, _):
        for c in range(_H // 16):
            ones[i, pl.ds(c * 16, 16)] = jnp.ones((16,), jnp.float32)
        return 0

    lax.fori_loop(0, _CHUNK, fill_ones, 0)

    def zero_bounce(i, _):
        for c in range(_H // 16):
            bounce[i, pl.ds(c * 16, 16)] = jnp.zeros((16,), jnp.float32)
        return 0

    lax.fori_loop(0, _WCH, zero_bounce, 0)

    def zero_acc(i, _):
        pltpu.sync_copy(bounce, acc.at[pl.ds(_row0(sid, i), _WCH)])
        return 0

    lax.fori_loop(0, _W_FULL, zero_acc, 0)

    @pl.when(sid < _W_EXTRA)
    def _():
        zero_acc(_W_FULL, 0)

    plsc.subcore_barrier()

    def step(k, _):
        off = _chunk_offset(cid, sid, k)
        pltpu.sync_copy(dst_hbm.at[pl.ds(off, _CHUNK)], didx)
        pltpu.sync_copy(ones, acc.at[didx], add=True)
        return 0

    lax.fori_loop(0, _FULL_IT, step, 0)

    @pl.when(sid < _EXTRA)
    def _():
        step(_FULL_IT, 0)

    plsc.subcore_barrier()

    def write_out(i, _):
        r0 = _row0(sid, i)
        pltpu.sync_copy(acc.at[pl.ds(r0, _WCH)], bounce)
        pltpu.sync_copy(bounce, out_hbm.at[cid, pl.ds(r0, _WCH)])
        return 0

    lax.fori_loop(0, _W_FULL, write_out, 0)

    @pl.when(sid < _W_EXTRA)
    def _():
        write_out(_W_FULL, 0)


@functools.partial(
    pl.kernel,
    out_type=jax.ShapeDtypeStruct((_NC, _N, _H), jnp.float32),
    mesh=_sc_mesh,
    scratch_types=[
        pltpu.VMEM((_CHUNK,), jnp.int32),
        pltpu.VMEM((_CHUNK,), jnp.int32),
        pltpu.VMEM((_CHUNK, _H), jnp.float32),
        pltpu.VMEM((_WCH, _H), jnp.float32),
        pltpu.VMEM_SHARED((_N, _H), jnp.float32),
        pltpu.SemaphoreType.DMA,
    ],
)
def _msg_kernel(src_hbm, dst_hbm, h_hbm, out_hbm, sidx, didx, rows, bounce, acc, sem):
    cid = lax.axis_index("c")
    sid = lax.axis_index("s")

    def zero_bounce(i, _):
        for c in range(_H // 16):
            bounce[i, pl.ds(c * 16, 16)] = jnp.zeros((16,), jnp.float32)
        return 0

    lax.fori_loop(0, _WCH, zero_bounce, 0)

    def zero_acc(i, _):
        pltpu.sync_copy(bounce, acc.at[pl.ds(_row0(sid, i), _WCH)])
        return 0

    lax.fori_loop(0, _W_FULL, zero_acc, 0)

    @pl.when(sid < _W_EXTRA)
    def _():
        zero_acc(_W_FULL, 0)

    plsc.subcore_barrier()

    def step(k, _):
        off = _chunk_offset(cid, sid, k)
        pltpu.sync_copy(src_hbm.at[pl.ds(off, _CHUNK)], sidx)
        pltpu.sync_copy(dst_hbm.at[pl.ds(off, _CHUNK)], didx)
        pltpu.async_copy(h_hbm.at[sidx], rows, sem).wait()
        pltpu.sync_copy(rows, acc.at[didx], add=True)
        return 0

    lax.fori_loop(0, _FULL_IT, step, 0)

    @pl.when(sid < _EXTRA)
    def _():
        step(_FULL_IT, 0)

    plsc.subcore_barrier()

    def write_out(i, _):
        r0 = _row0(sid, i)
        pltpu.sync_copy(acc.at[pl.ds(r0, _WCH)], bounce)
        pltpu.sync_copy(bounce, out_hbm.at[cid, pl.ds(r0, _WCH)])
        return 0

    lax.fori_loop(0, _W_FULL, write_out, 0)

    @pl.when(sid < _W_EXTRA)
    def _():
        write_out(_W_FULL, 0)


_R = 1000
_DOT = functools.partial(
    jnp.dot, precision=lax.Precision.HIGHEST, preferred_element_type=jnp.float32
)


def _tc_first(x, W1, degp):
    def body(x_ref, w_ref, degp_ref, h1p_ref, dinv_ref):
        deg = degp_ref[0] + degp_ref[1] + 1.0
        dinv = lax.rsqrt(jnp.maximum(deg, 1.0))
        dinv_ref[...] = dinv
        h = _DOT(x_ref[...], w_ref[...])
        h1p_ref[...] = h * dinv

    return pl.pallas_call(
        body,
        grid=(_N // _R,),
        in_specs=[
            pl.BlockSpec((_R, _D), lambda i: (i, 0)),
            pl.BlockSpec((_D, _H), lambda i: (0, 0)),
            pl.BlockSpec((_NC, _R, _H), lambda i: (0, i, 0)),
        ],
        out_specs=[
            pl.BlockSpec((_R, _H), lambda i: (i, 0)),
            pl.BlockSpec((_R, _H), lambda i: (i, 0)),
        ],
        out_shape=[
            jax.ShapeDtypeStruct((_N, _H), jnp.float32),
            jax.ShapeDtypeStruct((_N, _H), jnp.float32),
        ],
    )(x, W1, degp)


def _layer_post(p_ref, hp_ref, dinv, b_ref, g_ref, be_ref):
    agg = p_ref[0] + p_ref[1] + hp_ref[...]
    pre = agg * dinv + b_ref[...]
    m = jnp.mean(pre, axis=-1, keepdims=True)
    c = pre - m
    v = jnp.mean(c * c, axis=-1, keepdims=True)
    y = c * lax.rsqrt(v + 1e-5) * g_ref[...] + be_ref[...]
    return jnp.maximum(y, 0.0)


def _make_tc_layer(mode):
    def body(p_ref, hp_ref, dinv_ref, b_ref, g_ref, be_ref, w_ref, *rest):
        if mode == 1:
            xres_ref, x_out_ref, hn_ref = rest
        else:
            x_out_ref, hn_ref = rest
        dinv = dinv_ref[...]
        y = _layer_post(p_ref, hp_ref, dinv, b_ref, g_ref, be_ref)
        if mode == 1:
            y = y + 0.7 * rest[0][...]
        x_out_ref[...] = y
        hn_ref[...] = _DOT(y, w_ref[...]) * dinv

    in_specs = [
        pl.BlockSpec((_NC, _R, _H), lambda i: (0, i, 0)),
        pl.BlockSpec((_R, _H), lambda i: (i, 0)),
        pl.BlockSpec((_R, _H), lambda i: (i, 0)),
        pl.BlockSpec((1, _H), lambda i: (0, 0)),
        pl.BlockSpec((1, _H), lambda i: (0, 0)),
        pl.BlockSpec((1, _H), lambda i: (0, 0)),
        pl.BlockSpec((_H, _H), lambda i: (0, 0)),
    ]
    if mode == 1:
        in_specs.append(pl.BlockSpec((_R, _H), lambda i: (i, 0)))
    return pl.pallas_call(
        body,
        grid=(_N // _R,),
        in_specs=in_specs,
        out_specs=[
            pl.BlockSpec((_R, _H), lambda i: (i, 0)),
            pl.BlockSpec((_R, _H), lambda i: (i, 0)),
        ],
        out_shape=[
            jax.ShapeDtypeStruct((_N, _H), jnp.float32),
            jax.ShapeDtypeStruct((_N, _H), jnp.float32),
        ],
    )


def _tc_head(P3, h3p, dinv, b3, g3, be3, x2, fW1, fb1, fW2, fb2):
    def body(p_ref, hp_ref, dinv_ref, b_ref, g_ref, be_ref, xres_ref,
             fw1_ref, fb1_ref, fw2_ref, fb2_ref, out_ref):
        dinv = dinv_ref[...]
        y = _layer_post(p_ref, hp_ref, dinv, b_ref, g_ref, be_ref)
        x3 = y * 0.7 + xres_ref[...]
        h = jnp.maximum(_DOT(x3, fw1_ref[...]) + fb1_ref[...], 0.0)
        out_ref[...] = _DOT(h, fw2_ref[...]) + fb2_ref[...]

    return pl.pallas_call(
        body,
        grid=(_N // _R,),
        in_specs=[
            pl.BlockSpec((_NC, _R, _H), lambda i: (0, i, 0)),
            pl.BlockSpec((_R, _H), lambda i: (i, 0)),
            pl.BlockSpec((_R, _H), lambda i: (i, 0)),
            pl.BlockSpec((1, _H), lambda i: (0, 0)),
            pl.BlockSpec((1, _H), lambda i: (0, 0)),
            pl.BlockSpec((1, _H), lambda i: (0, 0)),
            pl.BlockSpec((_R, _H), lambda i: (i, 0)),
            pl.BlockSpec((_H, _H // 2), lambda i: (0, 0)),
            pl.BlockSpec((1, _H // 2), lambda i: (0, 0)),
            pl.BlockSpec((_H // 2, 1), lambda i: (0, 0)),
            pl.BlockSpec((1, 1), lambda i: (0, 0)),
        ],
        out_specs=pl.BlockSpec((_R, 1), lambda i: (i, 0)),
        out_shape=jax.ShapeDtypeStruct((_N, 1), jnp.float32),
    )(P3, h3p, dinv, b3, g3, be3, x2, fW1, fb1, fW2, fb2)


def kernel(x, edge_index, W1, b1, g1, be1, W2, b2, g2, be2, W3, b3, g3, be3,
           fW1, fb1, fW2, fb2):
    ei = edge_index.astype(jnp.int32)
    src, dst = ei[0], ei[1]
    r1 = lambda a: a.reshape(1, -1)

    degp = _deg_kernel(dst)
    h1p, dinv = _tc_first(x, W1, degp)
    P1 = _msg_kernel(src, dst, h1p)
    x1, h2p = _make_tc_layer(0)(P1, h1p, dinv, r1(b1), r1(g1), r1(be1), W2)
    P2 = _msg_kernel(src, dst, h2p)
    x2, h3p = _make_tc_layer(1)(P2, h2p, dinv, r1(b2), r1(g2), r1(be2), W3, x1)
    P3 = _msg_kernel(src, dst, h3p)
    return _tc_head(P3, h3p, dinv, r1(b3), r1(g3), r1(be3), x2,
                    fW1, r1(fb1), fW2.reshape(_H // 2, 1), fb2.reshape(1, 1))


# speedup vs baseline: 18.5224x; 1.4414x over previous
import functools

import jax
import jax.numpy as jnp
from jax import lax
from jax.experimental import pallas as pl
from jax.experimental.pallas import tpu as pltpu
from jax.experimental.pallas import tpu_sc as plsc

_N = 10000
_E = 320000
_D = 128
_H = 128

_NC = 2
_NS = 16

_CHUNK = 128
_NCHUNK = _E // _CHUNK
_CPC = _NCHUNK // _NC
_FULL_IT = _CPC // _NS
_EXTRA = _CPC - _NS * _FULL_IT

_WCH = 80
_NWCH = _N // _WCH
_W_FULL = _NWCH // _NS
_W_EXTRA = _NWCH - _NS * _W_FULL

_sc_mesh = plsc.VectorSubcoreMesh(core_axis_name="c", subcore_axis_name="s")


def _chunk_offset(cid, sid, k):
    return (cid * _CPC + sid + _NS * k) * _CHUNK


def _row0(sid, i):
    return (sid + _NS * i) * _WCH


@functools.partial(
    pl.kernel,
    out_type=jax.ShapeDtypeStruct((_NC, _N, _H), jnp.float32),
    mesh=_sc_mesh,
    scratch_types=[
        pltpu.VMEM((_CHUNK,), jnp.int32),
        pltpu.VMEM((_CHUNK, _H), jnp.float32),
        pltpu.VMEM((_WCH, _H), jnp.float32),
        pltpu.VMEM_SHARED((_N, _H), jnp.float32),
    ],
)
def _deg_kernel(dst_hbm, out_hbm, didx, ones, bounce, acc):
    cid = lax.axis_index("c")
    sid = lax.axis_index("s")

    def fill_ones(i, _):
        for c in range(_H // 16):
            ones[i, pl.ds(c * 16, 16)] = jnp.ones((16,), jnp.float32)
        return 0

    lax.fori_loop(0, _CHUNK, fill_ones, 0)

    def zero_bounce(i, _):
        for c in range(_H // 16):
            bounce[i, pl.ds(c * 16, 16)] = jnp.zeros((16,), jnp.float32)
        return 0

    lax.fori_loop(0, _WCH, zero_bounce, 0)

    def zero_acc(i, _):
        pltpu.sync_copy(bounce, acc.at[pl.ds(_row0(sid, i), _WCH)])
        return 0

    lax.fori_loop(0, _W_FULL, zero_acc, 0)

    @pl.when(sid < _W_EXTRA)
    def _():
        zero_acc(_W_FULL, 0)

    plsc.subcore_barrier()

    def step(k, _):
        off = _chunk_offset(cid, sid, k)
        pltpu.sync_copy(dst_hbm.at[pl.ds(off, _CHUNK)], didx)
        pltpu.sync_copy(ones, acc.at[didx], add=True)
        return 0

    lax.fori_loop(0, _FULL_IT, step, 0)

    @pl.when(sid < _EXTRA)
    def _():
        step(_FULL_IT, 0)

    plsc.subcore_barrier()

    def write_out(i, _):
        r0 = _row0(sid, i)
        pltpu.sync_copy(acc.at[pl.ds(r0, _WCH)], bounce)
        pltpu.sync_copy(bounce, out_hbm.at[cid, pl.ds(r0, _WCH)])
        return 0

    lax.fori_loop(0, _W_FULL, write_out, 0)

    @pl.when(sid < _W_EXTRA)
    def _():
        write_out(_W_FULL, 0)


@functools.partial(
    pl.kernel,
    out_type=jax.ShapeDtypeStruct((_NC, _N, _H), jnp.float32),
    mesh=_sc_mesh,
    scratch_types=[
        pltpu.VMEM((_CHUNK,), jnp.int32),
        pltpu.VMEM((_CHUNK,), jnp.int32),
        pltpu.VMEM((_CHUNK,), jnp.int32),
        pltpu.VMEM((_CHUNK,), jnp.int32),
        pltpu.VMEM((_CHUNK, _H), jnp.float32),
        pltpu.VMEM((_CHUNK, _H), jnp.float32),
        pltpu.VMEM((_WCH, _H), jnp.float32),
        pltpu.VMEM_SHARED((_N, _H), jnp.float32),
        pltpu.SemaphoreType.DMA,
        pltpu.SemaphoreType.DMA,
    ],
)
def _msg_kernel(src_hbm, dst_hbm, h_hbm, out_hbm,
                sidx0, sidx1, didx0, didx1, rows0, rows1,
                bounce, acc, gsem0, gsem1):
    cid = lax.axis_index("c")
    sid = lax.axis_index("s")
    sidx = (sidx0, sidx1)
    didx = (didx0, didx1)
    rows = (rows0, rows1)
    gsem = (gsem0, gsem1)

    def zero_bounce(i, _):
        for c in range(_H // 16):
            bounce[i, pl.ds(c * 16, 16)] = jnp.zeros((16,), jnp.float32)
        return 0

    lax.fori_loop(0, _WCH, zero_bounce, 0)

    def zero_acc(i, _):
        pltpu.sync_copy(bounce, acc.at[pl.ds(_row0(sid, i), _WCH)])
        return 0

    lax.fori_loop(0, _W_FULL, zero_acc, 0)

    @pl.when(sid < _W_EXTRA)
    def _():
        zero_acc(_W_FULL, 0)

    plsc.subcore_barrier()

    nk = _FULL_IT + jnp.where(sid < _EXTRA, 1, 0)

    def load_and_fire(b, k):
        off = _chunk_offset(cid, sid, k)
        pltpu.sync_copy(src_hbm.at[pl.ds(off, _CHUNK)], sidx[b])
        pltpu.sync_copy(dst_hbm.at[pl.ds(off, _CHUNK)], didx[b])
        pltpu.async_copy(h_hbm.at[sidx[b]], rows[b], gsem[b])

    def drain_and_scatter(b):
        pltpu.make_async_copy(h_hbm.at[sidx[b]], rows[b], gsem[b]).wait()
        pltpu.sync_copy(rows[b], acc.at[didx[b]], add=True)

    load_and_fire(0, 0)

    def group(g, _):
        k1 = 2 * g + 1

        @pl.when(k1 < nk)
        def _():
            load_and_fire(1, k1)

        drain_and_scatter(0)

        @pl.when(k1 + 1 < nk)
        def _():
            load_and_fire(0, k1 + 1)

        @pl.when(k1 < nk)
        def _():
            drain_and_scatter(1)

        return 0

    lax.fori_loop(0, (nk + 1) // 2, group, 0)

    plsc.subcore_barrier()

    def write_out(i, _):
        r0 = _row0(sid, i)
        pltpu.sync_copy(acc.at[pl.ds(r0, _WCH)], bounce)
        pltpu.sync_copy(bounce, out_hbm.at[cid, pl.ds(r0, _WCH)])
        return 0

    lax.fori_loop(0, _W_FULL, write_out, 0)

    @pl.when(sid < _W_EXTRA)
    def _():
        write_out(_W_FULL, 0)


_R = 1000
_DOT = functools.partial(
    jnp.dot, precision=lax.Precision.HIGHEST, preferred_element_type=jnp.float32
)


def _tc_first(x, W1, degp):
    def body(x_ref, w_ref, degp_ref, h1p_ref, dinv_ref):
        deg = degp_ref[0] + degp_ref[1] + 1.0
        dinv = lax.rsqrt(jnp.maximum(deg, 1.0))
        dinv_ref[...] = dinv
        h = _DOT(x_ref[...], w_ref[...])
        h1p_ref[...] = h * dinv

    return pl.pallas_call(
        body,
        grid=(_N // _R,),
        in_specs=[
            pl.BlockSpec((_R, _D), lambda i: (i, 0)),
            pl.BlockSpec((_D, _H), lambda i: (0, 0)),
            pl.BlockSpec((_NC, _R, _H), lambda i: (0, i, 0)),
        ],
        out_specs=[
            pl.BlockSpec((_R, _H), lambda i: (i, 0)),
            pl.BlockSpec((_R, _H), lambda i: (i, 0)),
        ],
        out_shape=[
            jax.ShapeDtypeStruct((_N, _H), jnp.float32),
            jax.ShapeDtypeStruct((_N, _H), jnp.float32),
        ],
    )(x, W1, degp)


def _layer_post(p_ref, hp_ref, dinv, b_ref, g_ref, be_ref):
    agg = p_ref[0] + p_ref[1] + hp_ref[...]
    pre = agg * dinv + b_ref[...]
    m = jnp.mean(pre, axis=-1, keepdims=True)
    c = pre - m
    v = jnp.mean(c * c, axis=-1, keepdims=True)
    y = c * lax.rsqrt(v + 1e-5) * g_ref[...] + be_ref[...]
    return jnp.maximum(y, 0.0)


def _make_tc_layer(mode):
    def body(p_ref, hp_ref, dinv_ref, b_ref, g_ref, be_ref, w_ref, *rest):
        if mode == 1:
            xres_ref, x_out_ref, hn_ref = rest
        else:
            x_out_ref, hn_ref = rest
        dinv = dinv_ref[...]
        y = _layer_post(p_ref, hp_ref, dinv, b_ref, g_ref, be_ref)
        if mode == 1:
            y = y + 0.7 * rest[0][...]
        x_out_ref[...] = y
        hn_ref[...] = _DOT(y, w_ref[...]) * dinv

    in_specs = [
        pl.BlockSpec((_NC, _R, _H), lambda i: (0, i, 0)),
        pl.BlockSpec((_R, _H), lambda i: (i, 0)),
        pl.BlockSpec((_R, _H), lambda i: (i, 0)),
        pl.BlockSpec((1, _H), lambda i: (0, 0)),
        pl.BlockSpec((1, _H), lambda i: (0, 0)),
        pl.BlockSpec((1, _H), lambda i: (0, 0)),
        pl.BlockSpec((_H, _H), lambda i: (0, 0)),
    ]
    if mode == 1:
        in_specs.append(pl.BlockSpec((_R, _H), lambda i: (i, 0)))
    return pl.pallas_call(
        body,
        grid=(_N // _R,),
        in_specs=in_specs,
        out_specs=[
            pl.BlockSpec((_R, _H), lambda i: (i, 0)),
            pl.BlockSpec((_R, _H), lambda i: (i, 0)),
        ],
        out_shape=[
            jax.ShapeDtypeStruct((_N, _H), jnp.float32),
            jax.ShapeDtypeStruct((_N, _H), jnp.float32),
        ],
    )


def _tc_head(P3, h3p, dinv, b3, g3, be3, x2, fW1, fb1, fW2, fb2):
    def body(p_ref, hp_ref, dinv_ref, b_ref, g_ref, be_ref, xres_ref,
             fw1_ref, fb1_ref, fw2_ref, fb2_ref, out_ref):
        dinv = dinv_ref[...]
        y = _layer_post(p_ref, hp_ref, dinv, b_ref, g_ref, be_ref)
        x3 = y * 0.7 + xres_ref[...]
        h = jnp.maximum(_DOT(x3, fw1_ref[...]) + fb1_ref[...], 0.0)
        out_ref[...] = _DOT(h, fw2_ref[...]) + fb2_ref[...]

    return pl.pallas_call(
        body,
        grid=(_N // _R,),
        in_specs=[
            pl.BlockSpec((_NC, _R, _H), lambda i: (0, i, 0)),
            pl.BlockSpec((_R, _H), lambda i: (i, 0)),
            pl.BlockSpec((_R, _H), lambda i: (i, 0)),
            pl.BlockSpec((1, _H), lambda i: (0, 0)),
            pl.BlockSpec((1, _H), lambda i: (0, 0)),
            pl.BlockSpec((1, _H), lambda i: (0, 0)),
            pl.BlockSpec((_R, _H), lambda i: (i, 0)),
            pl.BlockSpec((_H, _H // 2), lambda i: (0, 0)),
            pl.BlockSpec((1, _H // 2), lambda i: (0, 0)),
            pl.BlockSpec((_H // 2, 1), lambda i: (0, 0)),
            pl.BlockSpec((1, 1), lambda i: (0, 0)),
        ],
        out_specs=pl.BlockSpec((_R, 1), lambda i: (i, 0)),
        out_shape=jax.ShapeDtypeStruct((_N, 1), jnp.float32),
    )(P3, h3p, dinv, b3, g3, be3, x2, fW1, fb1, fW2, fb2)


def kernel(x, edge_index, W1, b1, g1, be1, W2, b2, g2, be2, W3, b3, g3, be3,
           fW1, fb1, fW2, fb2):
    ei = edge_index.astype(jnp.int32)
    src, dst = ei[0], ei[1]
    r1 = lambda a: a.reshape(1, -1)

    degp = _deg_kernel(dst)
    h1p, dinv = _tc_first(x, W1, degp)
    P1 = _msg_kernel(src, dst, h1p)
    x1, h2p = _make_tc_layer(0)(P1, h1p, dinv, r1(b1), r1(g1), r1(be1), W2)
    P2 = _msg_kernel(src, dst, h2p)
    x2, h3p = _make_tc_layer(1)(P2, h2p, dinv, r1(b2), r1(g2), r1(be2), W3, x1)
    P3 = _msg_kernel(src, dst, h3p)
    return _tc_head(P3, h3p, dinv, r1(b3), r1(g3), r1(be3), x2,
                    fW1, r1(fb1), fW2.reshape(_H // 2, 1), fb2.reshape(1, 1))


# speedup vs baseline: 22.2647x; 1.2020x over previous
import functools

import jax
import jax.numpy as jnp
from jax import lax
from jax.experimental import pallas as pl
from jax.experimental.pallas import tpu as pltpu
from jax.experimental.pallas import tpu_sc as plsc

_N = 10000
_E = 320000
_D = 128
_H = 128

_NC = 2
_NS = 16

_CHUNK = 80
_NCHUNK = _E // _CHUNK
_NW = _NC * _NS
_KPT = _NCHUNK // _NW
_PHASES = ((0, 64), (64, 61))
_IDXBUF = 64

_WCH = 80
_NWCH = _N // _WCH
_W_FULL = _NWCH // _NS
_W_EXTRA = _NWCH - _NS * _W_FULL

_sc_mesh = plsc.VectorSubcoreMesh(core_axis_name="c", subcore_axis_name="s")


def _row0(sid, i):
    return (sid + _NS * i) * _WCH


@functools.partial(
    pl.kernel,
    out_type=jax.ShapeDtypeStruct((_NC, _N, _H), jnp.float32),
    mesh=_sc_mesh,
    scratch_types=[
        pltpu.VMEM((_IDXBUF, 2, _CHUNK), jnp.int32),
        pltpu.VMEM((_CHUNK, _H), jnp.float32),
        pltpu.VMEM((_WCH, _H), jnp.float32),
        pltpu.VMEM_SHARED((_N, _H), jnp.float32),
        pltpu.SemaphoreType.DMA,
    ],
)
def _deg_kernel(eic_hbm, out_hbm, idxbuf, ones, bounce, acc, ssem):
    cid = lax.axis_index("c")
    sid = lax.axis_index("s")
    w = cid * _NS + sid

    def fill_ones(i, _):
        for c in range(_H // 16):
            ones[i, pl.ds(c * 16, 16)] = jnp.ones((16,), jnp.float32)
        return 0

    lax.fori_loop(0, _CHUNK, fill_ones, 0)

    def zero_bounce(i, _):
        for c in range(_H // 16):
            bounce[i, pl.ds(c * 16, 16)] = jnp.zeros((16,), jnp.float32)
        return 0

    lax.fori_loop(0, _WCH, zero_bounce, 0)

    def zero_acc(i, _):
        pltpu.sync_copy(bounce, acc.at[pl.ds(_row0(sid, i), _WCH)])
        return 0

    lax.fori_loop(0, _W_FULL, zero_acc, 0)

    @pl.when(sid < _W_EXTRA)
    def _():
        zero_acc(_W_FULL, 0)

    plsc.subcore_barrier()

    def fire(k):
        pltpu.async_copy(ones, acc.at[idxbuf.at[k, 1]], ssem, add=True)

    def drain(k):
        pltpu.make_async_copy(ones, acc.at[idxbuf.at[k, 1]], ssem).wait()

    for base, cnt in _PHASES:
        pltpu.sync_copy(
            eic_hbm.at[pl.ds(w * _KPT + base, cnt)], idxbuf.at[pl.ds(0, cnt)]
        )

        for j in range(4):
            fire(j)

        def qgroup(g, _):
            for j in range(4):
                k = 4 * g + 4 + j

                @pl.when(k < cnt)
                def _():
                    fire(k)

            for j in range(4):
                k = 4 * g + j

                @pl.when(k < cnt)
                def _():
                    drain(k)

            return 0

        lax.fori_loop(0, (cnt + 3) // 4, qgroup, 0)

    plsc.subcore_barrier()

    def write_out(i, _):
        r0 = _row0(sid, i)
        pltpu.sync_copy(acc.at[pl.ds(r0, _WCH)], bounce)
        pltpu.sync_copy(bounce, out_hbm.at[cid, pl.ds(r0, _WCH)])
        return 0

    lax.fori_loop(0, _W_FULL, write_out, 0)

    @pl.when(sid < _W_EXTRA)
    def _():
        write_out(_W_FULL, 0)


@functools.partial(
    pl.kernel,
    out_type=jax.ShapeDtypeStruct((_NC, _N, _H), jnp.float32),
    mesh=_sc_mesh,
    scratch_types=[
        pltpu.VMEM((_IDXBUF, 2, _CHUNK), jnp.int32),
        pltpu.VMEM((_CHUNK, _H), jnp.float32),
        pltpu.VMEM((_CHUNK, _H), jnp.float32),
        pltpu.VMEM_SHARED((_N, _H), jnp.float32),
        pltpu.SemaphoreType.DMA,
        pltpu.SemaphoreType.DMA,
        pltpu.SemaphoreType.DMA,
    ],
)
def _msg_kernel(eic_hbm, h_hbm, out_hbm,
                idxbuf, rows0, rows1, acc, gsem0, gsem1, ssem):
    cid = lax.axis_index("c")
    sid = lax.axis_index("s")
    rows = (rows0, rows1)
    gsem = (gsem0, gsem1)
    bounce = rows0
    w = cid * _NS + sid

    def zero_bounce(i, _):
        for c in range(_H // 16):
            bounce[i, pl.ds(c * 16, 16)] = jnp.zeros((16,), jnp.float32)
        return 0

    lax.fori_loop(0, _WCH, zero_bounce, 0)

    def zero_acc(i, _):
        pltpu.sync_copy(bounce, acc.at[pl.ds(_row0(sid, i), _WCH)])
        return 0

    lax.fori_loop(0, _W_FULL, zero_acc, 0)

    @pl.when(sid < _W_EXTRA)
    def _():
        zero_acc(_W_FULL, 0)

    plsc.subcore_barrier()

    def fire_gather(b, k):
        pltpu.async_copy(h_hbm.at[idxbuf.at[k, 0]], rows[b], gsem[b])

    def wait_gather(b, k):
        pltpu.make_async_copy(h_hbm.at[idxbuf.at[k, 0]], rows[b], gsem[b]).wait()

    def fire_scatter(b, k):
        pltpu.async_copy(rows[b], acc.at[idxbuf.at[k, 1]], ssem, add=True)

    def wait_scatter(b, k):
        pltpu.make_async_copy(rows[b], acc.at[idxbuf.at[k, 1]], ssem).wait()

    for base, cnt in _PHASES:
        pltpu.sync_copy(
            eic_hbm.at[pl.ds(w * _KPT + base, cnt)], idxbuf.at[pl.ds(0, cnt)]
        )

        fire_gather(0, 0)

        def group(g, _):
            k0 = 2 * g
            k1 = 2 * g + 1

            @pl.when(k1 < cnt)
            def _():
                fire_gather(1, k1)

            wait_gather(0, k0)
            fire_scatter(0, k0)

            @pl.when(k1 + 1 < cnt)
            def _():
                wait_scatter(0, k0)
                fire_gather(0, k1 + 1)

            @pl.when(k1 < cnt)
            def _():
                wait_gather(1, k1)
                fire_scatter(1, k1)
                wait_scatter(1, k1)

            return 0

        lax.fori_loop(0, (cnt + 1) // 2, group, 0)

        wait_scatter(0, 2 * ((cnt + 1) // 2) - 2)

    plsc.subcore_barrier()

    def write_out(i, _):
        r0 = _row0(sid, i)
        pltpu.sync_copy(acc.at[pl.ds(r0, _WCH)], bounce)
        pltpu.sync_copy(bounce, out_hbm.at[cid, pl.ds(r0, _WCH)])
        return 0

    lax.fori_loop(0, _W_FULL, write_out, 0)

    @pl.when(sid < _W_EXTRA)
    def _():
        write_out(_W_FULL, 0)


_R = 1000
_DOT = functools.partial(
    jnp.dot, precision=lax.Precision.HIGHEST, preferred_element_type=jnp.float32
)


def _tc_first(x, W1, degp):
    def body(x_ref, w_ref, degp_ref, h1p_ref, dinv_ref):
        deg = degp_ref[0] + degp_ref[1] + 1.0
        dinv = lax.rsqrt(jnp.maximum(deg, 1.0))
        dinv_ref[...] = dinv
        h = _DOT(x_ref[...], w_ref[...])
        h1p_ref[...] = h * dinv

    return pl.pallas_call(
        body,
        grid=(_N // _R,),
        in_specs=[
            pl.BlockSpec((_R, _D), lambda i: (i, 0)),
            pl.BlockSpec((_D, _H), lambda i: (0, 0)),
            pl.BlockSpec((_NC, _R, _H), lambda i: (0, i, 0)),
        ],
        out_specs=[
            pl.BlockSpec((_R, _H), lambda i: (i, 0)),
            pl.BlockSpec((_R, _H), lambda i: (i, 0)),
        ],
        out_shape=[
            jax.ShapeDtypeStruct((_N, _H), jnp.float32),
            jax.ShapeDtypeStruct((_N, _H), jnp.float32),
        ],
    )(x, W1, degp)


def _layer_post(p_ref, hp_ref, dinv, b_ref, g_ref, be_ref):
    agg = p_ref[0] + p_ref[1] + hp_ref[...]
    pre = agg * dinv + b_ref[...]
    m = jnp.mean(pre, axis=-1, keepdims=True)
    c = pre - m
    v = jnp.mean(c * c, axis=-1, keepdims=True)
    y = c * lax.rsqrt(v + 1e-5) * g_ref[...] + be_ref[...]
    return jnp.maximum(y, 0.0)


def _make_tc_layer(mode):
    def body(p_ref, hp_ref, dinv_ref, b_ref, g_ref, be_ref, w_ref, *rest):
        if mode == 1:
            xres_ref, x_out_ref, hn_ref = rest
        else:
            x_out_ref, hn_ref = rest
        dinv = dinv_ref[...]
        y = _layer_post(p_ref, hp_ref, dinv, b_ref, g_ref, be_ref)
        if mode == 1:
            y = y + 0.7 * rest[0][...]
        x_out_ref[...] = y
        hn_ref[...] = _DOT(y, w_ref[...]) * dinv

    in_specs = [
        pl.BlockSpec((_NC, _R, _H), lambda i: (0, i, 0)),
        pl.BlockSpec((_R, _H), lambda i: (i, 0)),
        pl.BlockSpec((_R, _H), lambda i: (i, 0)),
        pl.BlockSpec((1, _H), lambda i: (0, 0)),
        pl.BlockSpec((1, _H), lambda i: (0, 0)),
        pl.BlockSpec((1, _H), lambda i: (0, 0)),
        pl.BlockSpec((_H, _H), lambda i: (0, 0)),
    ]
    if mode == 1:
        in_specs.append(pl.BlockSpec((_R, _H), lambda i: (i, 0)))
    return pl.pallas_call(
        body,
        grid=(_N // _R,),
        in_specs=in_specs,
        out_specs=[
            pl.BlockSpec((_R, _H), lambda i: (i, 0)),
            pl.BlockSpec((_R, _H), lambda i: (i, 0)),
        ],
        out_shape=[
            jax.ShapeDtypeStruct((_N, _H), jnp.float32),
            jax.ShapeDtypeStruct((_N, _H), jnp.float32),
        ],
    )


def _tc_head(P3, h3p, dinv, b3, g3, be3, x2, fW1, fb1, fW2, fb2):
    def body(p_ref, hp_ref, dinv_ref, b_ref, g_ref, be_ref, xres_ref,
             fw1_ref, fb1_ref, fw2_ref, fb2_ref, out_ref):
        dinv = dinv_ref[...]
        y = _layer_post(p_ref, hp_ref, dinv, b_ref, g_ref, be_ref)
        x3 = y * 0.7 + xres_ref[...]
        h = jnp.maximum(_DOT(x3, fw1_ref[...]) + fb1_ref[...], 0.0)
        out_ref[...] = _DOT(h, fw2_ref[...]) + fb2_ref[...]

    return pl.pallas_call(
        body,
        grid=(_N // _R,),
        in_specs=[
            pl.BlockSpec((_NC, _R, _H), lambda i: (0, i, 0)),
            pl.BlockSpec((_R, _H), lambda i: (i, 0)),
            pl.BlockSpec((_R, _H), lambda i: (i, 0)),
            pl.BlockSpec((1, _H), lambda i: (0, 0)),
            pl.BlockSpec((1, _H), lambda i: (0, 0)),
            pl.BlockSpec((1, _H), lambda i: (0, 0)),
            pl.BlockSpec((_R, _H), lambda i: (i, 0)),
            pl.BlockSpec((_H, _H // 2), lambda i: (0, 0)),
            pl.BlockSpec((1, _H // 2), lambda i: (0, 0)),
            pl.BlockSpec((_H // 2, 1), lambda i: (0, 0)),
            pl.BlockSpec((1, 1), lambda i: (0, 0)),
        ],
        out_specs=pl.BlockSpec((_R, 1), lambda i: (i, 0)),
        out_shape=jax.ShapeDtypeStruct((_N, 1), jnp.float32),
    )(P3, h3p, dinv, b3, g3, be3, x2, fW1, fb1, fW2, fb2)


def kernel(x, edge_index, W1, b1, g1, be1, W2, b2, g2, be2, W3, b3, g3, be3,
           fW1, fb1, fW2, fb2):
    ei = edge_index.astype(jnp.int32)
    eic = ei.reshape(2, _NCHUNK, _CHUNK).swapaxes(0, 1)
    r1 = lambda a: a.reshape(1, -1)

    degp = _deg_kernel(eic)
    h1p, dinv = _tc_first(x, W1, degp)
    P1 = _msg_kernel(eic, h1p)
    x1, h2p = _make_tc_layer(0)(P1, h1p, dinv, r1(b1), r1(g1), r1(be1), W2)
    P2 = _msg_kernel(eic, h2p)
    x2, h3p = _make_tc_layer(1)(P2, h2p, dinv, r1(b2), r1(g2), r1(be2), W3, x1)
    P3 = _msg_kernel(eic, h3p)
    return _tc_head(P3, h3p, dinv, r1(b3), r1(g3), r1(be3), x2,
                    fW1, r1(fb1), fW2.reshape(_H // 2, 1), fb2.reshape(1, 1))


# speedup vs baseline: 23.5873x; 1.0594x over previous
import functools

import jax
import jax.numpy as jnp
from jax import lax
from jax.experimental import pallas as pl
from jax.experimental.pallas import tpu as pltpu
from jax.experimental.pallas import tpu_sc as plsc

_N = 10000
_E = 320000
_D = 128
_H = 128

_NC = 2
_NS = 16

_CHUNK = 40
_NCHUNK = _E // _CHUNK
_NW = _NC * _NS
_KPT = _NCHUNK // _NW
_IDXBUF = 50
_PHASES = tuple((i * _IDXBUF, _IDXBUF) for i in range(_KPT // _IDXBUF))

_WCH = 40
_NWCH = _N // _WCH
_W_FULL = _NWCH // _NS
_W_EXTRA = _NWCH - _NS * _W_FULL

_sc_mesh = plsc.VectorSubcoreMesh(core_axis_name="c", subcore_axis_name="s")


def _row0(sid, i):
    return (sid + _NS * i) * _WCH


@functools.partial(
    pl.kernel,
    out_type=jax.ShapeDtypeStruct((_NC, _N, _H), jnp.float32),
    mesh=_sc_mesh,
    scratch_types=[
        pltpu.VMEM((_IDXBUF, 2, _CHUNK), jnp.int32),
        pltpu.VMEM((_CHUNK, _H), jnp.float32),
        pltpu.VMEM((_WCH, _H), jnp.float32),
        pltpu.VMEM_SHARED((_N, _H), jnp.float32),
        pltpu.SemaphoreType.DMA,
    ],
)
def _deg_kernel(eic_hbm, out_hbm, idxbuf, ones, bounce, acc, ssem):
    cid = lax.axis_index("c")
    sid = lax.axis_index("s")
    w = cid * _NS + sid

    def fill_ones(i, _):
        for c in range(_H // 16):
            ones[i, pl.ds(c * 16, 16)] = jnp.ones((16,), jnp.float32)
        return 0

    lax.fori_loop(0, _CHUNK, fill_ones, 0)

    def zero_bounce(i, _):
        for c in range(_H // 16):
            bounce[i, pl.ds(c * 16, 16)] = jnp.zeros((16,), jnp.float32)
        return 0

    lax.fori_loop(0, _WCH, zero_bounce, 0)

    def zero_acc(i, _):
        pltpu.sync_copy(bounce, acc.at[pl.ds(_row0(sid, i), _WCH)])
        return 0

    lax.fori_loop(0, _W_FULL, zero_acc, 0)

    @pl.when(sid < _W_EXTRA)
    def _():
        zero_acc(_W_FULL, 0)

    plsc.subcore_barrier()

    def fire(k):
        pltpu.async_copy(ones, acc.at[idxbuf.at[k, 1]], ssem, add=True)

    def drain(k):
        pltpu.make_async_copy(ones, acc.at[idxbuf.at[k, 1]], ssem).wait()

    for base, cnt in _PHASES:
        pltpu.sync_copy(
            eic_hbm.at[pl.ds(w * _KPT + base, cnt)], idxbuf.at[pl.ds(0, cnt)]
        )

        for j in range(4):
            fire(j)

        def qgroup(g, _):
            for j in range(4):
                k = 4 * g + 4 + j

                @pl.when(k < cnt)
                def _():
                    fire(k)

            for j in range(4):
                k = 4 * g + j

                @pl.when(k < cnt)
                def _():
                    drain(k)

            return 0

        lax.fori_loop(0, (cnt + 3) // 4, qgroup, 0)

    plsc.subcore_barrier()

    def write_out(i, _):
        r0 = _row0(sid, i)
        pltpu.sync_copy(acc.at[pl.ds(r0, _WCH)], bounce)
        pltpu.sync_copy(bounce, out_hbm.at[cid, pl.ds(r0, _WCH)])
        return 0

    lax.fori_loop(0, _W_FULL, write_out, 0)

    @pl.when(sid < _W_EXTRA)
    def _():
        write_out(_W_FULL, 0)


@functools.partial(
    pl.kernel,
    out_type=jax.ShapeDtypeStruct((_NC, _N, _H), jnp.float32),
    mesh=_sc_mesh,
    scratch_types=[
        pltpu.VMEM((_IDXBUF, 2, _CHUNK), jnp.int32),
        pltpu.VMEM((_CHUNK, _H), jnp.float32),
        pltpu.VMEM((_CHUNK, _H), jnp.float32),
        pltpu.VMEM((_CHUNK, _H), jnp.float32),
        pltpu.VMEM((_CHUNK, _H), jnp.float32),
        pltpu.VMEM_SHARED((_N, _H), jnp.float32),
        pltpu.SemaphoreType.DMA,
        pltpu.SemaphoreType.DMA,
        pltpu.SemaphoreType.DMA,
        pltpu.SemaphoreType.DMA,
        pltpu.SemaphoreType.DMA,
    ],
)
def _msg_kernel(eic_hbm, h_hbm, out_hbm,
                idxbuf, rows0, rows1, rows2, rows3, acc,
                gsem0, gsem1, gsem2, gsem3, ssem):
    cid = lax.axis_index("c")
    sid = lax.axis_index("s")
    rows = (rows0, rows1, rows2, rows3)
    gsem = (gsem0, gsem1, gsem2, gsem3)
    bounce = rows0
    w = cid * _NS + sid

    def zero_bounce(i, _):
        for c in range(_H // 16):
            bounce[i, pl.ds(c * 16, 16)] = jnp.zeros((16,), jnp.float32)
        return 0

    lax.fori_loop(0, _WCH, zero_bounce, 0)

    def zero_acc(i, _):
        pltpu.sync_copy(bounce, acc.at[pl.ds(_row0(sid, i), _WCH)])
        return 0

    lax.fori_loop(0, _W_FULL, zero_acc, 0)

    @pl.when(sid < _W_EXTRA)
    def _():
        zero_acc(_W_FULL, 0)

    plsc.subcore_barrier()

    def fire_gather(b, k):
        pltpu.async_copy(h_hbm.at[idxbuf.at[k, 0]], rows[b], gsem[b])

    def wait_gather(b, k):
        pltpu.make_async_copy(h_hbm.at[idxbuf.at[k, 0]], rows[b], gsem[b]).wait()

    def fire_scatter(b, k):
        pltpu.async_copy(rows[b], acc.at[idxbuf.at[k, 1]], ssem, add=True)

    def wait_scatter(b, k):
        pltpu.make_async_copy(rows[b], acc.at[idxbuf.at[k, 1]], ssem).wait()

    for base, cnt in _PHASES:
        pltpu.sync_copy(
            eic_hbm.at[pl.ds(w * _KPT + base, cnt)], idxbuf.at[pl.ds(0, cnt)]
        )

        fire_gather(0, 0)
        fire_gather(1, 1)

        def group(g, _):
            for b in range(4):
                k = 4 * g + b
                bb = (b + 2) % 4

                @pl.when(k < cnt)
                def _():
                    @pl.when(k >= 2)
                    def _():
                        wait_scatter(bb, k - 2)

                    @pl.when(k + 2 < cnt)
                    def _():
                        fire_gather(bb, k + 2)

                    wait_gather(b, k)
                    fire_scatter(b, k)

            return 0

        lax.fori_loop(0, (cnt + 3) // 4, group, 0)

        for k in range(max(0, cnt - 2), cnt):
            wait_scatter(k % 4, k)

    plsc.subcore_barrier()

    def write_out(i, _):
        r0 = _row0(sid, i)
        pltpu.sync_copy(acc.at[pl.ds(r0, _WCH)], bounce)
        pltpu.sync_copy(bounce, out_hbm.at[cid, pl.ds(r0, _WCH)])
        return 0

    lax.fori_loop(0, _W_FULL, write_out, 0)

    @pl.when(sid < _W_EXTRA)
    def _():
        write_out(_W_FULL, 0)


_R = 1000
_DOT = functools.partial(jnp.dot, preferred_element_type=jnp.float32)


def _tc_first(x, W1, degp):
    def body(x_ref, w_ref, degp_ref, h1p_ref, dinv_ref):
        deg = degp_ref[0] + degp_ref[1] + 1.0
        dinv = lax.rsqrt(jnp.maximum(deg, 1.0))
        dinv_ref[...] = dinv
        h = _DOT(x_ref[...], w_ref[...])
        h1p_ref[...] = h * dinv

    return pl.pallas_call(
        body,
        grid=(_N // _R,),
        in_specs=[
            pl.BlockSpec((_R, _D), lambda i: (i, 0)),
            pl.BlockSpec((_D, _H), lambda i: (0, 0)),
            pl.BlockSpec((_NC, _R, _H), lambda i: (0, i, 0)),
        ],
        out_specs=[
            pl.BlockSpec((_R, _H), lambda i: (i, 0)),
            pl.BlockSpec((_R, _H), lambda i: (i, 0)),
        ],
        out_shape=[
            jax.ShapeDtypeStruct((_N, _H), jnp.float32),
            jax.ShapeDtypeStruct((_N, _H), jnp.float32),
        ],
    )(x, W1, degp)


def _layer_post(p_ref, hp_ref, dinv, b_ref, g_ref, be_ref):
    agg = p_ref[0] + p_ref[1] + hp_ref[...]
    pre = agg * dinv + b_ref[...]
    m = jnp.mean(pre, axis=-1, keepdims=True)
    c = pre - m
    v = jnp.mean(c * c, axis=-1, keepdims=True)
    y = c * lax.rsqrt(v + 1e-5) * g_ref[...] + be_ref[...]
    return jnp.maximum(y, 0.0)


def _make_tc_layer(mode):
    def body(p_ref, hp_ref, dinv_ref, b_ref, g_ref, be_ref, w_ref, *rest):
        if mode == 1:
            xres_ref, x_out_ref, hn_ref = rest
        else:
            x_out_ref, hn_ref = rest
        dinv = dinv_ref[...]
        y = _layer_post(p_ref, hp_ref, dinv, b_ref, g_ref, be_ref)
        if mode == 1:
            y = y + 0.7 * rest[0][...]
        x_out_ref[...] = y
        hn_ref[...] = _DOT(y, w_ref[...]) * dinv

    in_specs = [
        pl.BlockSpec((_NC, _R, _H), lambda i: (0, i, 0)),
        pl.BlockSpec((_R, _H), lambda i: (i, 0)),
        pl.BlockSpec((_R, _H), lambda i: (i, 0)),
        pl.BlockSpec((1, _H), lambda i: (0, 0)),
        pl.BlockSpec((1, _H), lambda i: (0, 0)),
        pl.BlockSpec((1, _H), lambda i: (0, 0)),
        pl.BlockSpec((_H, _H), lambda i: (0, 0)),
    ]
    if mode == 1:
        in_specs.append(pl.BlockSpec((_R, _H), lambda i: (i, 0)))
    return pl.pallas_call(
        body,
        grid=(_N // _R,),
        in_specs=in_specs,
        out_specs=[
            pl.BlockSpec((_R, _H), lambda i: (i, 0)),
            pl.BlockSpec((_R, _H), lambda i: (i, 0)),
        ],
        out_shape=[
            jax.ShapeDtypeStruct((_N, _H), jnp.float32),
            jax.ShapeDtypeStruct((_N, _H), jnp.float32),
        ],
    )


def _tc_head(P3, h3p, dinv, b3, g3, be3, x2, fW1, fb1, fW2, fb2):
    def body(p_ref, hp_ref, dinv_ref, b_ref, g_ref, be_ref, xres_ref,
             fw1_ref, fb1_ref, fw2_ref, fb2_ref, out_ref):
        dinv = dinv_ref[...]
        y = _layer_post(p_ref, hp_ref, dinv, b_ref, g_ref, be_ref)
        x3 = y * 0.7 + xres_ref[...]
        h = jnp.maximum(_DOT(x3, fw1_ref[...]) + fb1_ref[...], 0.0)
        out_ref[...] = _DOT(h, fw2_ref[...]) + fb2_ref[...]

    return pl.pallas_call(
        body,
        grid=(_N // _R,),
        in_specs=[
            pl.BlockSpec((_NC, _R, _H), lambda i: (0, i, 0)),
            pl.BlockSpec((_R, _H), lambda i: (i, 0)),
            pl.BlockSpec((_R, _H), lambda i: (i, 0)),
            pl.BlockSpec((1, _H), lambda i: (0, 0)),
            pl.BlockSpec((1, _H), lambda i: (0, 0)),
            pl.BlockSpec((1, _H), lambda i: (0, 0)),
            pl.BlockSpec((_R, _H), lambda i: (i, 0)),
            pl.BlockSpec((_H, _H // 2), lambda i: (0, 0)),
            pl.BlockSpec((1, _H // 2), lambda i: (0, 0)),
            pl.BlockSpec((_H // 2, 1), lambda i: (0, 0)),
            pl.BlockSpec((1, 1), lambda i: (0, 0)),
        ],
        out_specs=pl.BlockSpec((_R, 1), lambda i: (i, 0)),
        out_shape=jax.ShapeDtypeStruct((_N, 1), jnp.float32),
    )(P3, h3p, dinv, b3, g3, be3, x2, fW1, fb1, fW2, fb2)


def kernel(x, edge_index, W1, b1, g1, be1, W2, b2, g2, be2, W3, b3, g3, be3,
           fW1, fb1, fW2, fb2):
    ei = edge_index.astype(jnp.int32)
    eic = ei.reshape(2, _NCHUNK, _CHUNK).swapaxes(0, 1)
    r1 = lambda a: a.reshape(1, -1)

    degp = _deg_kernel(eic)
    h1p, dinv = _tc_first(x, W1, degp)
    P1 = _msg_kernel(eic, h1p)
    x1, h2p = _make_tc_layer(0)(P1, h1p, dinv, r1(b1), r1(g1), r1(be1), W2)
    P2 = _msg_kernel(eic, h2p)
    x2, h3p = _make_tc_layer(1)(P2, h2p, dinv, r1(b2), r1(g2), r1(be2), W3, x1)
    P3 = _msg_kernel(eic, h3p)
    return _tc_head(P3, h3p, dinv, r1(b3), r1(g3), r1(be3), x2,
                    fW1, r1(fb1), fW2.reshape(_H // 2, 1), fb2.reshape(1, 1))


# speedup vs baseline: 25.6698x; 1.0883x over previous
import functools

import jax
import jax.numpy as jnp
from jax import lax
from jax.experimental import pallas as pl
from jax.experimental.pallas import tpu as pltpu
from jax.experimental.pallas import tpu_sc as plsc

_N = 10000
_E = 320000
_D = 128
_H = 128

_NC = 2
_NS = 16

_CHUNK = 40
_NCHUNK = _E // _CHUNK
_NW = _NC * _NS
_KPT = _NCHUNK // _NW
_IDXBUF = 50
_PHASES = tuple((i * _IDXBUF, _IDXBUF) for i in range(_KPT // _IDXBUF))

_WCH = 40
_NWCH = _N // _WCH
_W_FULL = _NWCH // _NS
_W_EXTRA = _NWCH - _NS * _W_FULL

_sc_mesh = plsc.VectorSubcoreMesh(core_axis_name="c", subcore_axis_name="s")


def _row0(sid, i):
    return (sid + _NS * i) * _WCH


@functools.partial(
    pl.kernel,
    out_type=jax.ShapeDtypeStruct((_NC, _N), jnp.float32),
    mesh=_sc_mesh,
    scratch_types=[
        pltpu.VMEM((_IDXBUF, 2, _CHUNK), jnp.int32),
        pltpu.VMEM((64,), jnp.float32),
        pltpu.VMEM((_N,), jnp.float32),
        pltpu.VMEM_SHARED((_N,), jnp.float32),
        pltpu.SemaphoreType.DMA,
    ],
)
def _deg_kernel(eic_hbm, out_hbm, idxbuf, ones, bounce, acc, ssem):
    cid = lax.axis_index("c")
    sid = lax.axis_index("s")
    w = cid * _NS + sid

    for c in range(4):
        ones[pl.ds(c * 16, 16)] = jnp.ones((16,), jnp.float32)

    @pl.when(sid == 0)
    def _():
        def zb(i, _):
            bounce[pl.ds(i * 16, 16)] = jnp.zeros((16,), jnp.float32)
            return 0

        lax.fori_loop(0, _N // 16, zb, 0)
        pltpu.sync_copy(bounce, acc)

    plsc.subcore_barrier()

    def fire(k):
        pltpu.async_copy(ones.at[pl.ds(0, _CHUNK)], acc.at[idxbuf.at[k, 1]],
                         ssem, add=True)

    def drain(k):
        pltpu.make_async_copy(ones.at[pl.ds(0, _CHUNK)],
                              acc.at[idxbuf.at[k, 1]], ssem).wait()

    for base, cnt in _PHASES:
        pltpu.sync_copy(
            eic_hbm.at[pl.ds(w * _KPT + base, cnt)], idxbuf.at[pl.ds(0, cnt)]
        )

        for j in range(4):
            fire(j)

        def qgroup(g, _):
            for j in range(4):
                k = 4 * g + 4 + j

                @pl.when(k < cnt)
                def _():
                    fire(k)

            for j in range(4):
                k = 4 * g + j

                @pl.when(k < cnt)
                def _():
                    drain(k)

            return 0

        lax.fori_loop(0, (cnt + 3) // 4, qgroup, 0)

    plsc.subcore_barrier()

    @pl.when(sid == 0)
    def _():
        pltpu.sync_copy(acc, out_hbm.at[cid])


@functools.partial(
    pl.kernel,
    out_type=jax.ShapeDtypeStruct((_NC, _N, _H), jnp.float32),
    mesh=_sc_mesh,
    scratch_types=[
        pltpu.VMEM((_IDXBUF, 2, _CHUNK), jnp.int32),
        pltpu.VMEM((_CHUNK, _H), jnp.float32),
        pltpu.VMEM((_CHUNK, _H), jnp.float32),
        pltpu.VMEM((_CHUNK, _H), jnp.float32),
        pltpu.VMEM((_CHUNK, _H), jnp.float32),
        pltpu.VMEM_SHARED((_N, _H), jnp.float32),
        pltpu.SemaphoreType.DMA,
        pltpu.SemaphoreType.DMA,
        pltpu.SemaphoreType.DMA,
        pltpu.SemaphoreType.DMA,
        pltpu.SemaphoreType.DMA,
    ],
)
def _msg_kernel(eic_hbm, h_hbm, out_hbm,
                idxbuf, rows0, rows1, rows2, rows3, acc,
                gsem0, gsem1, gsem2, gsem3, ssem):
    cid = lax.axis_index("c")
    sid = lax.axis_index("s")
    rows = (rows0, rows1, rows2, rows3)
    gsem = (gsem0, gsem1, gsem2, gsem3)
    bounce = rows0
    w = cid * _NS + sid

    def zero_bounce(i, _):
        for c in range(_H // 16):
            bounce[i, pl.ds(c * 16, 16)] = jnp.zeros((16,), jnp.float32)
        return 0

    lax.fori_loop(0, _WCH, zero_bounce, 0)

    def zero_acc(i, _):
        pltpu.sync_copy(bounce, acc.at[pl.ds(_row0(sid, i), _WCH)])
        return 0

    lax.fori_loop(0, _W_FULL, zero_acc, 0)

    @pl.when(sid < _W_EXTRA)
    def _():
        zero_acc(_W_FULL, 0)

    plsc.subcore_barrier()

    def fire_gather(b, k):
        pltpu.async_copy(h_hbm.at[idxbuf.at[k, 0]], rows[b], gsem[b])

    def wait_gather(b, k):
        pltpu.make_async_copy(h_hbm.at[idxbuf.at[k, 0]], rows[b], gsem[b]).wait()

    def fire_scatter(b, k):
        pltpu.async_copy(rows[b], acc.at[idxbuf.at[k, 1]], ssem, add=True)

    def wait_scatter(b, k):
        pltpu.make_async_copy(rows[b], acc.at[idxbuf.at[k, 1]], ssem).wait()

    for base, cnt in _PHASES:
        pltpu.sync_copy(
            eic_hbm.at[pl.ds(w * _KPT + base, cnt)], idxbuf.at[pl.ds(0, cnt)]
        )

        fire_gather(0, 0)
        fire_gather(1, 1)

        def group(g, _):
            for b in range(4):
                k = 4 * g + b
                bb = (b + 2) % 4

                @pl.when(k < cnt)
                def _():
                    @pl.when(k >= 2)
                    def _():
                        wait_scatter(bb, k - 2)

                    @pl.when(k + 2 < cnt)
                    def _():
                        fire_gather(bb, k + 2)

                    wait_gather(b, k)
                    fire_scatter(b, k)

            return 0

        lax.fori_loop(0, (cnt + 3) // 4, group, 0)

        for k in range(max(0, cnt - 2), cnt):
            wait_scatter(k % 4, k)

    plsc.subcore_barrier()

    def write_out(i, _):
        r0 = _row0(sid, i)
        pltpu.sync_copy(acc.at[pl.ds(r0, _WCH)], out_hbm.at[cid, pl.ds(r0, _WCH)])
        return 0

    lax.fori_loop(0, _W_FULL, write_out, 0)

    @pl.when(sid < _W_EXTRA)
    def _():
        write_out(_W_FULL, 0)


_R = 1000
_DOT = functools.partial(jnp.dot, preferred_element_type=jnp.float32)


def _tc_first(x, W1, dinv):
    def body(x_ref, w_ref, dinv_ref, h1p_ref):
        h1p_ref[...] = _DOT(x_ref[...], w_ref[...]) * dinv_ref[...]

    return pl.pallas_call(
        body,
        grid=(_N // _R,),
        in_specs=[
            pl.BlockSpec((_R, _D), lambda i: (i, 0)),
            pl.BlockSpec((_D, _H), lambda i: (0, 0)),
            pl.BlockSpec((_R, 1), lambda i: (i, 0)),
        ],
        out_specs=pl.BlockSpec((_R, _H), lambda i: (i, 0)),
        out_shape=jax.ShapeDtypeStruct((_N, _H), jnp.float32),
    )(x, W1, dinv)


def _layer_post(p_ref, hp_ref, dinv, b_ref, g_ref, be_ref):
    agg = p_ref[0] + p_ref[1] + hp_ref[...]
    pre = agg * dinv + b_ref[...]
    m = jnp.mean(pre, axis=-1, keepdims=True)
    c = pre - m
    v = jnp.mean(c * c, axis=-1, keepdims=True)
    y = c * lax.rsqrt(v + 1e-5) * g_ref[...] + be_ref[...]
    return jnp.maximum(y, 0.0)


def _make_tc_layer(mode):
    def body(p_ref, hp_ref, dinv_ref, b_ref, g_ref, be_ref, w_ref, *rest):
        if mode == 1:
            xres_ref, x_out_ref, hn_ref = rest
        else:
            x_out_ref, hn_ref = rest
        dinv = dinv_ref[...]
        y = _layer_post(p_ref, hp_ref, dinv, b_ref, g_ref, be_ref)
        if mode == 1:
            y = y + 0.7 * rest[0][...]
        x_out_ref[...] = y
        hn_ref[...] = _DOT(y, w_ref[...]) * dinv

    in_specs = [
        pl.BlockSpec((_NC, _R, _H), lambda i: (0, i, 0)),
        pl.BlockSpec((_R, _H), lambda i: (i, 0)),
        pl.BlockSpec((_R, 1), lambda i: (i, 0)),
        pl.BlockSpec((1, _H), lambda i: (0, 0)),
        pl.BlockSpec((1, _H), lambda i: (0, 0)),
        pl.BlockSpec((1, _H), lambda i: (0, 0)),
        pl.BlockSpec((_H, _H), lambda i: (0, 0)),
    ]
    if mode == 1:
        in_specs.append(pl.BlockSpec((_R, _H), lambda i: (i, 0)))
    return pl.pallas_call(
        body,
        grid=(_N // _R,),
        in_specs=in_specs,
        out_specs=[
            pl.BlockSpec((_R, _H), lambda i: (i, 0)),
            pl.BlockSpec((_R, _H), lambda i: (i, 0)),
        ],
        out_shape=[
            jax.ShapeDtypeStruct((_N, _H), jnp.float32),
            jax.ShapeDtypeStruct((_N, _H), jnp.float32),
        ],
    )


def _tc_head(P3, h3p, dinv, b3, g3, be3, x2, fW1, fb1, fW2, fb2):
    def body(p_ref, hp_ref, dinv_ref, b_ref, g_ref, be_ref, xres_ref,
             fw1_ref, fb1_ref, fw2_ref, fb2_ref, out_ref):
        dinv = dinv_ref[...]
        y = _layer_post(p_ref, hp_ref, dinv, b_ref, g_ref, be_ref)
        x3 = y * 0.7 + xres_ref[...]
        h = jnp.maximum(_DOT(x3, fw1_ref[...]) + fb1_ref[...], 0.0)
        out_ref[...] = _DOT(h, fw2_ref[...]) + fb2_ref[...]

    return pl.pallas_call(
        body,
        grid=(_N // _R,),
        in_specs=[
            pl.BlockSpec((_NC, _R, _H), lambda i: (0, i, 0)),
            pl.BlockSpec((_R, _H), lambda i: (i, 0)),
            pl.BlockSpec((_R, 1), lambda i: (i, 0)),
            pl.BlockSpec((1, _H), lambda i: (0, 0)),
            pl.BlockSpec((1, _H), lambda i: (0, 0)),
            pl.BlockSpec((1, _H), lambda i: (0, 0)),
            pl.BlockSpec((_R, _H), lambda i: (i, 0)),
            pl.BlockSpec((_H, _H // 2), lambda i: (0, 0)),
            pl.BlockSpec((1, _H // 2), lambda i: (0, 0)),
            pl.BlockSpec((_H // 2, 1), lambda i: (0, 0)),
            pl.BlockSpec((1, 1), lambda i: (0, 0)),
        ],
        out_specs=pl.BlockSpec((_R, 1), lambda i: (i, 0)),
        out_shape=jax.ShapeDtypeStruct((_N, 1), jnp.float32),
    )(P3, h3p, dinv, b3, g3, be3, x2, fW1, fb1, fW2, fb2)


def kernel(x, edge_index, W1, b1, g1, be1, W2, b2, g2, be2, W3, b3, g3, be3,
           fW1, fb1, fW2, fb2):
    ei = edge_index.astype(jnp.int32)
    eic = ei.reshape(2, _NCHUNK, _CHUNK).swapaxes(0, 1)
    r1 = lambda a: a.reshape(1, -1)

    degp = _deg_kernel(eic)
    dinv = lax.rsqrt(jnp.maximum(degp[0] + degp[1] + 1.0, 1.0))[:, None]
    h1p = _tc_first(x, W1, dinv)
    P1 = _msg_kernel(eic, h1p)
    x1, h2p = _make_tc_layer(0)(P1, h1p, dinv, r1(b1), r1(g1), r1(be1), W2)
    P2 = _msg_kernel(eic, h2p)
    x2, h3p = _make_tc_layer(1)(P2, h2p, dinv, r1(b2), r1(g2), r1(be2), W3, x1)
    P3 = _msg_kernel(eic, h3p)
    return _tc_head(P3, h3p, dinv, r1(b3), r1(g3), r1(be3), x2,
                    fW1, r1(fb1), fW2.reshape(_H // 2, 1), fb2.reshape(1, 1))


# speedup vs baseline: 26.4623x; 1.0309x over previous
import functools

import jax
import jax.numpy as jnp
from jax import lax
from jax.experimental import pallas as pl
from jax.experimental.pallas import tpu as pltpu
from jax.experimental.pallas import tpu_sc as plsc

_N = 10000
_E = 320000
_D = 128
_H = 128

_NC = 2
_NS = 16

_CHUNK = 40
_NCHUNK = _E // _CHUNK
_NW = _NC * _NS
_KPT = _NCHUNK // _NW
_IDXBUF = 84
_PHASES = ((0, 84), (84, 83), (167, 83))

_WCH = 40
_NWCH = _N // _WCH
_W_FULL = _NWCH // _NS
_W_EXTRA = _NWCH - _NS * _W_FULL

_sc_mesh = plsc.VectorSubcoreMesh(core_axis_name="c", subcore_axis_name="s")


def _row0(sid, i):
    return (sid + _NS * i) * _WCH


@functools.partial(
    pl.kernel,
    out_type=jax.ShapeDtypeStruct((_NC, _N), jnp.float32),
    mesh=_sc_mesh,
    scratch_types=[
        pltpu.VMEM((_IDXBUF, 2, _CHUNK), jnp.int32),
        pltpu.VMEM((64,), jnp.float32),
        pltpu.VMEM((_N,), jnp.float32),
        pltpu.VMEM_SHARED((_N,), jnp.float32),
        pltpu.SemaphoreType.DMA,
    ],
)
def _deg_kernel(eic_hbm, out_hbm, idxbuf, ones, bounce, acc, ssem):
    cid = lax.axis_index("c")
    sid = lax.axis_index("s")
    w = cid * _NS + sid

    for c in range(4):
        ones[pl.ds(c * 16, 16)] = jnp.ones((16,), jnp.float32)

    @pl.when(sid == 0)
    def _():
        def zb(i, _):
            bounce[pl.ds(i * 16, 16)] = jnp.zeros((16,), jnp.float32)
            return 0

        lax.fori_loop(0, _N // 16, zb, 0)
        pltpu.sync_copy(bounce, acc)

    plsc.subcore_barrier()

    def fire(k):
        pltpu.async_copy(ones.at[pl.ds(0, _CHUNK)], acc.at[idxbuf.at[k, 1]],
                         ssem, add=True)

    def drain(k):
        pltpu.make_async_copy(ones.at[pl.ds(0, _CHUNK)],
                              acc.at[idxbuf.at[k, 1]], ssem).wait()

    for base, cnt in _PHASES:
        pltpu.sync_copy(
            eic_hbm.at[pl.ds(w * _KPT + base, cnt)], idxbuf.at[pl.ds(0, cnt)]
        )

        for j in range(4):
            fire(j)

        def qgroup(g, _):
            for j in range(4):
                k = 4 * g + 4 + j

                @pl.when(k < cnt)
                def _():
                    fire(k)

            for j in range(4):
                k = 4 * g + j

                @pl.when(k < cnt)
                def _():
                    drain(k)

            return 0

        lax.fori_loop(0, (cnt + 3) // 4, qgroup, 0)

    plsc.subcore_barrier()

    @pl.when(sid == 0)
    def _():
        pltpu.sync_copy(acc, out_hbm.at[cid])


@functools.partial(
    pl.kernel,
    out_type=jax.ShapeDtypeStruct((_NC, _N, _H), jnp.float32),
    mesh=_sc_mesh,
    scratch_types=[
        pltpu.VMEM((_IDXBUF, 2, _CHUNK), jnp.int32),
        pltpu.VMEM((_CHUNK, _H), jnp.float32),
        pltpu.VMEM((_CHUNK, _H), jnp.float32),
        pltpu.VMEM((_CHUNK, _H), jnp.float32),
        pltpu.VMEM((_CHUNK, _H), jnp.float32),
        pltpu.VMEM_SHARED((_N, _H), jnp.float32),
        pltpu.SemaphoreType.DMA,
        pltpu.SemaphoreType.DMA,
        pltpu.SemaphoreType.DMA,
        pltpu.SemaphoreType.DMA,
        pltpu.SemaphoreType.DMA,
    ],
)
def _msg_kernel(eic_hbm, h_hbm, out_hbm,
                idxbuf, rows0, rows1, rows2, rows3, acc,
                gsem0, gsem1, gsem2, gsem3, ssem):
    cid = lax.axis_index("c")
    sid = lax.axis_index("s")
    rows = (rows0, rows1, rows2, rows3)
    gsem = (gsem0, gsem1, gsem2, gsem3)
    bounce = rows2
    w = cid * _NS + sid

    def fire_gather(b, k):
        pltpu.async_copy(h_hbm.at[idxbuf.at[k, 0]], rows[b], gsem[b])

    def wait_gather(b, k):
        pltpu.make_async_copy(h_hbm.at[idxbuf.at[k, 0]], rows[b], gsem[b]).wait()

    def fire_scatter(b, k):
        pltpu.async_copy(rows[b], acc.at[idxbuf.at[k, 1]], ssem, add=True)

    def wait_scatter(b, k):
        pltpu.make_async_copy(rows[b], acc.at[idxbuf.at[k, 1]], ssem).wait()

    pltpu.sync_copy(
        eic_hbm.at[pl.ds(w * _KPT, _PHASES[0][1])],
        idxbuf.at[pl.ds(0, _PHASES[0][1])],
    )
    fire_gather(0, 0)
    fire_gather(1, 1)

    def zero_bounce(i, _):
        for c in range(_H // 16):
            bounce[i, pl.ds(c * 16, 16)] = jnp.zeros((16,), jnp.float32)
        return 0

    lax.fori_loop(0, _WCH, zero_bounce, 0)

    def zero_acc(i, _):
        pltpu.sync_copy(bounce, acc.at[pl.ds(_row0(sid, i), _WCH)])
        return 0

    lax.fori_loop(0, _W_FULL, zero_acc, 0)

    @pl.when(sid < _W_EXTRA)
    def _():
        zero_acc(_W_FULL, 0)

    plsc.subcore_barrier()

    for pi, (base, cnt) in enumerate(_PHASES):
        if pi > 0:
            pltpu.sync_copy(
                eic_hbm.at[pl.ds(w * _KPT + base, cnt)], idxbuf.at[pl.ds(0, cnt)]
            )
            fire_gather(0, 0)
            fire_gather(1, 1)

        def group(g, _):
            for b in range(4):
                k = 4 * g + b
                bb = (b + 2) % 4

                @pl.when(k < cnt)
                def _():
                    @pl.when(k >= 2)
                    def _():
                        wait_scatter(bb, k - 2)

                    @pl.when(k + 2 < cnt)
                    def _():
                        fire_gather(bb, k + 2)

                    wait_gather(b, k)
                    fire_scatter(b, k)

            return 0

        lax.fori_loop(0, (cnt + 3) // 4, group, 0)

        for k in range(max(0, cnt - 2), cnt):
            wait_scatter(k % 4, k)

    plsc.subcore_barrier()

    def write_out(i, _):
        r0 = _row0(sid, i)
        pltpu.sync_copy(acc.at[pl.ds(r0, _WCH)], out_hbm.at[cid, pl.ds(r0, _WCH)])
        return 0

    lax.fori_loop(0, _W_FULL, write_out, 0)

    @pl.when(sid < _W_EXTRA)
    def _():
        write_out(_W_FULL, 0)


_R = 1000
_DOT = functools.partial(jnp.dot, preferred_element_type=jnp.float32)


def _tc_first(x, W1, dinv):
    def body(x_ref, w_ref, dinv_ref, h1p_ref):
        h1p_ref[...] = _DOT(x_ref[...], w_ref[...]) * dinv_ref[...]

    return pl.pallas_call(
        body,
        grid=(_N // _R,),
        in_specs=[
            pl.BlockSpec((_R, _D), lambda i: (i, 0)),
            pl.BlockSpec((_D, _H), lambda i: (0, 0)),
            pl.BlockSpec((_R, 1), lambda i: (i, 0)),
        ],
        out_specs=pl.BlockSpec((_R, _H), lambda i: (i, 0)),
        out_shape=jax.ShapeDtypeStruct((_N, _H), jnp.float32),
    )(x, W1, dinv)


def _layer_post(p_ref, hp_ref, dinv, b_ref, g_ref, be_ref):
    agg = p_ref[0] + p_ref[1] + hp_ref[...]
    pre = agg * dinv + b_ref[...]
    m = jnp.mean(pre, axis=-1, keepdims=True)
    c = pre - m
    v = jnp.mean(c * c, axis=-1, keepdims=True)
    y = c * lax.rsqrt(v + 1e-5) * g_ref[...] + be_ref[...]
    return jnp.maximum(y, 0.0)


def _make_tc_layer(mode):
    def body(p_ref, hp_ref, dinv_ref, b_ref, g_ref, be_ref, w_ref, *rest):
        if mode == 1:
            xres_ref, x_out_ref, hn_ref = rest
        else:
            x_out_ref, hn_ref = rest
        dinv = dinv_ref[...]
        y = _layer_post(p_ref, hp_ref, dinv, b_ref, g_ref, be_ref)
        if mode == 1:
            y = y + 0.7 * rest[0][...]
        x_out_ref[...] = y
        hn_ref[...] = _DOT(y, w_ref[...]) * dinv

    in_specs = [
        pl.BlockSpec((_NC, _R, _H), lambda i: (0, i, 0)),
        pl.BlockSpec((_R, _H), lambda i: (i, 0)),
        pl.BlockSpec((_R, 1), lambda i: (i, 0)),
        pl.BlockSpec((1, _H), lambda i: (0, 0)),
        pl.BlockSpec((1, _H), lambda i: (0, 0)),
        pl.BlockSpec((1, _H), lambda i: (0, 0)),
        pl.BlockSpec((_H, _H), lambda i: (0, 0)),
    ]
    if mode == 1:
        in_specs.append(pl.BlockSpec((_R, _H), lambda i: (i, 0)))
    return pl.pallas_call(
        body,
        grid=(_N // _R,),
        in_specs=in_specs,
        out_specs=[
            pl.BlockSpec((_R, _H), lambda i: (i, 0)),
            pl.BlockSpec((_R, _H), lambda i: (i, 0)),
        ],
        out_shape=[
            jax.ShapeDtypeStruct((_N, _H), jnp.float32),
            jax.ShapeDtypeStruct((_N, _H), jnp.float32),
        ],
    )


def _tc_head(P3, h3p, dinv, b3, g3, be3, x2, fW1, fb1, fW2, fb2):
    def body(p_ref, hp_ref, dinv_ref, b_ref, g_ref, be_ref, xres_ref,
             fw1_ref, fb1_ref, fw2_ref, fb2_ref, out_ref):
        dinv = dinv_ref[...]
        y = _layer_post(p_ref, hp_ref, dinv, b_ref, g_ref, be_ref)
        x3 = y * 0.7 + xres_ref[...]
        h = jnp.maximum(_DOT(x3, fw1_ref[...]) + fb1_ref[...], 0.0)
        out_ref[...] = _DOT(h, fw2_ref[...]) + fb2_ref[...]

    return pl.pallas_call(
        body,
        grid=(_N // _R,),
        in_specs=[
            pl.BlockSpec((_NC, _R, _H), lambda i: (0, i, 0)),
            pl.BlockSpec((_R, _H), lambda i: (i, 0)),
            pl.BlockSpec((_R, 1), lambda i: (i, 0)),
            pl.BlockSpec((1, _H), lambda i: (0, 0)),
            pl.BlockSpec((1, _H), lambda i: (0, 0)),
            pl.BlockSpec((1, _H), lambda i: (0, 0)),
            pl.BlockSpec((_R, _H), lambda i: (i, 0)),
            pl.BlockSpec((_H, _H // 2), lambda i: (0, 0)),
            pl.BlockSpec((1, _H // 2), lambda i: (0, 0)),
            pl.BlockSpec((_H // 2, 1), lambda i: (0, 0)),
            pl.BlockSpec((1, 1), lambda i: (0, 0)),
        ],
        out_specs=pl.BlockSpec((_R, 1), lambda i: (i, 0)),
        out_shape=jax.ShapeDtypeStruct((_N, 1), jnp.float32),
    )(P3, h3p, dinv, b3, g3, be3, x2, fW1, fb1, fW2, fb2)


def kernel(x, edge_index, W1, b1, g1, be1, W2, b2, g2, be2, W3, b3, g3, be3,
           fW1, fb1, fW2, fb2):
    ei = edge_index.astype(jnp.int32)
    eic = ei.reshape(2, _NCHUNK, _CHUNK).swapaxes(0, 1)
    r1 = lambda a: a.reshape(1, -1)

    degp = _deg_kernel(eic)
    dinv = lax.rsqrt(jnp.maximum(degp[0] + degp[1] + 1.0, 1.0))[:, None]
    h1p = _tc_first(x, W1, dinv)
    P1 = _msg_kernel(eic, h1p)
    x1, h2p = _make_tc_layer(0)(P1, h1p, dinv, r1(b1), r1(g1), r1(be1), W2)
    P2 = _msg_kernel(eic, h2p)
    x2, h3p = _make_tc_layer(1)(P2, h2p, dinv, r1(b2), r1(g2), r1(be2), W3, x1)
    P3 = _msg_kernel(eic, h3p)
    return _tc_head(P3, h3p, dinv, r1(b3), r1(g3), r1(be3), x2,
                    fW1, r1(fb1), fW2.reshape(_H // 2, 1), fb2.reshape(1, 1))


# speedup vs baseline: 27.6680x; 1.0456x over previous
import functools

import jax
import jax.numpy as jnp
from jax import lax
from jax.experimental import pallas as pl
from jax.experimental.pallas import tpu as pltpu
from jax.experimental.pallas import tpu_sc as plsc

_N = 10000
_E = 320000
_D = 128
_H = 128

_NC = 2
_NS = 16

_CHUNK = 40
_NCHUNK = _E // _CHUNK
_NW = _NC * _NS
_KPT = _NCHUNK // _NW
_IDXBUF = 84
_PHASES = ((0, 84), (84, 83), (167, 83))

_WCH = 40
_NWCH = _N // _WCH
_W_FULL = _NWCH // _NS
_W_EXTRA = _NWCH - _NS * _W_FULL

_sc_mesh = plsc.VectorSubcoreMesh(core_axis_name="c", subcore_axis_name="s")


def _row0(sid, i):
    return (sid + _NS * i) * _WCH


_DCH = 80
_DKPT = _E // _NW // _DCH


@functools.partial(
    pl.kernel,
    out_type=jax.ShapeDtypeStruct((_NC, _N), jnp.float32),
    mesh=_sc_mesh,
    scratch_types=[
        pltpu.VMEM((_DKPT, _DCH), jnp.int32),
        pltpu.VMEM((_DCH,), jnp.float32),
        pltpu.VMEM((_N,), jnp.float32),
        pltpu.VMEM_SHARED((_N,), jnp.float32),
        pltpu.SemaphoreType.DMA,
    ],
)
def _deg_kernel(dstc_hbm, out_hbm, didxbuf, ones, bounce, acc, ssem):
    cid = lax.axis_index("c")
    sid = lax.axis_index("s")
    w = cid * _NS + sid

    pltpu.sync_copy(dstc_hbm.at[w], didxbuf)

    for c in range(_DCH // 16):
        ones[pl.ds(c * 16, 16)] = jnp.ones((16,), jnp.float32)

    @pl.when(sid == 0)
    def _():
        def zb(i, _):
            bounce[pl.ds(i * 16, 16)] = jnp.zeros((16,), jnp.float32)
            return 0

        lax.fori_loop(0, _N // 16, zb, 0)
        pltpu.sync_copy(bounce, acc)

    plsc.subcore_barrier()

    def fire(k):
        pltpu.async_copy(ones, acc.at[didxbuf.at[k]], ssem, add=True)

    def drain(k):
        pltpu.make_async_copy(ones, acc.at[didxbuf.at[k]], ssem).wait()

    for j in range(4):
        fire(j)

    def qgroup(g, _):
        for j in range(4):
            k = 4 * g + 4 + j

            @pl.when(k < _DKPT)
            def _():
                fire(k)

        for j in range(4):
            k = 4 * g + j

            @pl.when(k < _DKPT)
            def _():
                drain(k)

        return 0

    lax.fori_loop(0, (_DKPT + 3) // 4, qgroup, 0)

    plsc.subcore_barrier()

    @pl.when(sid == 0)
    def _():
        pltpu.sync_copy(acc, out_hbm.at[cid])


@functools.partial(
    pl.kernel,
    out_type=jax.ShapeDtypeStruct((_NC, _N, _H), jnp.float32),
    mesh=_sc_mesh,
    scratch_types=[
        pltpu.VMEM((_IDXBUF, 1, _CHUNK), jnp.int32),
        pltpu.VMEM((_IDXBUF, 1, _CHUNK), jnp.int32),
        pltpu.VMEM((_CHUNK, _H), jnp.float32),
        pltpu.VMEM((_CHUNK, _H), jnp.float32),
        pltpu.VMEM((_CHUNK, _H), jnp.float32),
        pltpu.VMEM((_CHUNK, _H), jnp.float32),
        pltpu.VMEM_SHARED((_N, _H), jnp.float32),
        pltpu.SemaphoreType.DMA,
        pltpu.SemaphoreType.DMA,
        pltpu.SemaphoreType.DMA,
        pltpu.SemaphoreType.DMA,
        pltpu.SemaphoreType.DMA,
    ],
)
def _msg_kernel(srcc_hbm, dstc_hbm, h_hbm, out_hbm,
                sidxbuf, didxbuf, rows0, rows1, rows2, rows3, acc,
                gsem0, gsem1, gsem2, gsem3, ssem):
    cid = lax.axis_index("c")
    sid = lax.axis_index("s")
    rows = (rows0, rows1, rows2, rows3)
    gsem = (gsem0, gsem1, gsem2, gsem3)
    bounce = rows2
    w = cid * _NS + sid

    def fire_gather(b, k):
        pltpu.async_copy(h_hbm.at[sidxbuf.at[k, 0]], rows[b], gsem[b])

    def wait_gather(b, k):
        pltpu.make_async_copy(h_hbm.at[sidxbuf.at[k, 0]], rows[b], gsem[b]).wait()

    def fire_scatter(b, k):
        pltpu.async_copy(rows[b], acc.at[didxbuf.at[k, 0]], ssem, add=True)

    def wait_scatter(b, k):
        pltpu.make_async_copy(rows[b], acc.at[didxbuf.at[k, 0]], ssem).wait()

    def preload(base, cnt):
        pltpu.sync_copy(
            srcc_hbm.at[pl.ds(w * _KPT + base, cnt)], sidxbuf.at[pl.ds(0, cnt)]
        )
        pltpu.sync_copy(
            dstc_hbm.at[pl.ds(w * _KPT + base, cnt)], didxbuf.at[pl.ds(0, cnt)]
        )

    preload(*_PHASES[0])
    fire_gather(0, 0)
    fire_gather(1, 1)

    def zero_bounce(i, _):
        for c in range(_H // 16):
            bounce[i, pl.ds(c * 16, 16)] = jnp.zeros((16,), jnp.float32)
        return 0

    lax.fori_loop(0, _WCH, zero_bounce, 0)

    def zero_acc(i, _):
        pltpu.sync_copy(bounce, acc.at[pl.ds(_row0(sid, i), _WCH)])
        return 0

    lax.fori_loop(0, _W_FULL, zero_acc, 0)

    @pl.when(sid < _W_EXTRA)
    def _():
        zero_acc(_W_FULL, 0)

    plsc.subcore_barrier()

    for pi, (base, cnt) in enumerate(_PHASES):
        if pi > 0:
            preload(base, cnt)
            fire_gather(0, 0)
            fire_gather(1, 1)

        def group(g, _):
            for b in range(4):
                k = 4 * g + b
                bb = (b + 2) % 4

                @pl.when(k < cnt)
                def _():
                    @pl.when(k >= 2)
                    def _():
                        wait_scatter(bb, k - 2)

                    @pl.when(k + 2 < cnt)
                    def _():
                        fire_gather(bb, k + 2)

                    wait_gather(b, k)
                    fire_scatter(b, k)

            return 0

        lax.fori_loop(0, (cnt + 3) // 4, group, 0)

        for k in range(max(0, cnt - 2), cnt):
            wait_scatter(k % 4, k)

    plsc.subcore_barrier()

    def write_out(i, _):
        r0 = _row0(sid, i)
        pltpu.sync_copy(acc.at[pl.ds(r0, _WCH)], out_hbm.at[cid, pl.ds(r0, _WCH)])
        return 0

    lax.fori_loop(0, _W_FULL, write_out, 0)

    @pl.when(sid < _W_EXTRA)
    def _():
        write_out(_W_FULL, 0)


_R = 1000
_DOT = functools.partial(jnp.dot, preferred_element_type=jnp.float32)


def _tc_first(x, W1, dinv):
    def body(x_ref, w_ref, dinv_ref, h1p_ref):
        h1p_ref[...] = _DOT(x_ref[...], w_ref[...]) * dinv_ref[...]

    return pl.pallas_call(
        body,
        grid=(_N // _R,),
        in_specs=[
            pl.BlockSpec((_R, _D), lambda i: (i, 0)),
            pl.BlockSpec((_D, _H), lambda i: (0, 0)),
            pl.BlockSpec((_R, 1), lambda i: (i, 0)),
        ],
        out_specs=pl.BlockSpec((_R, _H), lambda i: (i, 0)),
        out_shape=jax.ShapeDtypeStruct((_N, _H), jnp.float32),
    )(x, W1, dinv)


def _layer_post(p_ref, hp_ref, dinv, b_ref, g_ref, be_ref):
    agg = p_ref[0] + p_ref[1] + hp_ref[...]
    pre = agg * dinv + b_ref[...]
    m = jnp.mean(pre, axis=-1, keepdims=True)
    c = pre - m
    v = jnp.mean(c * c, axis=-1, keepdims=True)
    y = c * lax.rsqrt(v + 1e-5) * g_ref[...] + be_ref[...]
    return jnp.maximum(y, 0.0)


def _make_tc_layer(mode):
    def body(p_ref, hp_ref, dinv_ref, b_ref, g_ref, be_ref, w_ref, *rest):
        if mode == 1:
            xres_ref, x_out_ref, hn_ref = rest
        else:
            x_out_ref, hn_ref = rest
        dinv = dinv_ref[...]
        y = _layer_post(p_ref, hp_ref, dinv, b_ref, g_ref, be_ref)
        if mode == 1:
            y = y + 0.7 * rest[0][...]
        x_out_ref[...] = y
        hn_ref[...] = _DOT(y, w_ref[...]) * dinv

    in_specs = [
        pl.BlockSpec((_NC, _R, _H), lambda i: (0, i, 0)),
        pl.BlockSpec((_R, _H), lambda i: (i, 0)),
        pl.BlockSpec((_R, 1), lambda i: (i, 0)),
        pl.BlockSpec((1, _H), lambda i: (0, 0)),
        pl.BlockSpec((1, _H), lambda i: (0, 0)),
        pl.BlockSpec((1, _H), lambda i: (0, 0)),
        pl.BlockSpec((_H, _H), lambda i: (0, 0)),
    ]
    if mode == 1:
        in_specs.append(pl.BlockSpec((_R, _H), lambda i: (i, 0)))
    return pl.pallas_call(
        body,
        grid=(_N // _R,),
        in_specs=in_specs,
        out_specs=[
            pl.BlockSpec((_R, _H), lambda i: (i, 0)),
            pl.BlockSpec((_R, _H), lambda i: (i, 0)),
        ],
        out_shape=[
            jax.ShapeDtypeStruct((_N, _H), jnp.float32),
            jax.ShapeDtypeStruct((_N, _H), jnp.float32),
        ],
    )


def _tc_head(P3, h3p, dinv, b3, g3, be3, x2, fW1, fb1, fW2, fb2):
    def body(p_ref, hp_ref, dinv_ref, b_ref, g_ref, be_ref, xres_ref,
             fw1_ref, fb1_ref, fw2_ref, fb2_ref, out_ref):
        dinv = dinv_ref[...]
        y = _layer_post(p_ref, hp_ref, dinv, b_ref, g_ref, be_ref)
        x3 = y * 0.7 + xres_ref[...]
        h = jnp.maximum(_DOT(x3, fw1_ref[...]) + fb1_ref[...], 0.0)
        out_ref[...] = _DOT(h, fw2_ref[...]) + fb2_ref[...]

    return pl.pallas_call(
        body,
        grid=(_N // _R,),
        in_specs=[
            pl.BlockSpec((_NC, _R, _H), lambda i: (0, i, 0)),
            pl.BlockSpec((_R, _H), lambda i: (i, 0)),
            pl.BlockSpec((_R, 1), lambda i: (i, 0)),
            pl.BlockSpec((1, _H), lambda i: (0, 0)),
            pl.BlockSpec((1, _H), lambda i: (0, 0)),
            pl.BlockSpec((1, _H), lambda i: (0, 0)),
            pl.BlockSpec((_R, _H), lambda i: (i, 0)),
            pl.BlockSpec((_H, _H // 2), lambda i: (0, 0)),
            pl.BlockSpec((1, _H // 2), lambda i: (0, 0)),
            pl.BlockSpec((_H // 2, 1), lambda i: (0, 0)),
            pl.BlockSpec((1, 1), lambda i: (0, 0)),
        ],
        out_specs=pl.BlockSpec((_R, 1), lambda i: (i, 0)),
        out_shape=jax.ShapeDtypeStruct((_N, 1), jnp.float32),
    )(P3, h3p, dinv, b3, g3, be3, x2, fW1, fb1, fW2, fb2)


def kernel(x, edge_index, W1, b1, g1, be1, W2, b2, g2, be2, W3, b3, g3, be3,
           fW1, fb1, fW2, fb2):
    ei = edge_index.astype(jnp.int32)
    srcc = ei[0].reshape(_NCHUNK, 1, _CHUNK)
    dstc = ei[1].reshape(_NCHUNK, 1, _CHUNK)
    dstc80 = ei[1].reshape(_NW, _DKPT, _DCH)
    r1 = lambda a: a.reshape(1, -1)

    degp = _deg_kernel(dstc80)
    dinv = lax.rsqrt(jnp.maximum(degp[0] + degp[1] + 1.0, 1.0))[:, None]
    h1p = _tc_first(x, W1, dinv)
    P1 = _msg_kernel(srcc, dstc, h1p)
    x1, h2p = _make_tc_layer(0)(P1, h1p, dinv, r1(b1), r1(g1), r1(be1), W2)
    P2 = _msg_kernel(srcc, dstc, h2p)
    x2, h3p = _make_tc_layer(1)(P2, h2p, dinv, r1(b2), r1(g2), r1(be2), W3, x1)
    P3 = _msg_kernel(srcc, dstc, h3p)
    return _tc_head(P3, h3p, dinv, r1(b3), r1(g3), r1(be3), x2,
                    fW1, r1(fb1), fW2.reshape(_H // 2, 1), fb2.reshape(1, 1))
